# Initial kernel scaffold; baseline (speedup 1.0000x reference)
import jax
import jax.numpy as jnp
from jax import lax
from jax.experimental import pallas as pl
from jax.experimental.pallas import tpu as pltpu
from jax.experimental.pallas import tpu_sc as plsc

RC = 5.0
NB = 4
NC = 2
NS = 16
NWK = NC * NS
WIN = 128

_HI = lax.Precision.HIGHEST


def _dot(a, b):
    return jnp.dot(a, b, precision=_HI, preferred_element_type=jnp.float32)


def _edge_tc(p1ci, p1cj, p3j, diff, pi_Wa, pi_Wb, pi_b2, W2a, W2b, W2c):
    e, d = p1ci.shape
    be = 6400
    assert e % be == 0
    has_p3 = p3j is not None

    def body(*refs):
        if has_p3:
            (p1ci_r, p1cj_r, p3j0_r, p3j1_r, p3j2_r, diff_r,
             wa_r, wb_r, b_r, w2a_r, w2b_r, w2c_r,
             o2_r, o30_r, o31_r, o32_r) = refs
            p3rs = (p3j0_r, p3j1_r, p3j2_r)
        else:
            (p1ci_r, p1cj_r, diff_r,
             wa_r, wb_r, b_r, w2a_r, w2b_r, w2c_r,
             o2_r, o30_r, o31_r, o32_r) = refs
        diffb = diff_r[...]
        dist = jnp.sqrt(jnp.sum(diffb * diffb, axis=1, keepdims=True) + 1e-12)
        fc = jnp.where(dist < RC, 0.5 * (jnp.cos(jnp.pi / RC * dist) + 1.0), 0.0)
        col = lax.broadcasted_iota(jnp.int32, (be, d * NB), 1) % NB
        fcb = jnp.broadcast_to(fc, (be, d * NB))
        fc2 = fcb * fcb
        basis_t = jnp.where(col == 0, fcb,
                  jnp.where(col == 1, fc2,
                  jnp.where(col == 2, fc2 * fcb, fc2 * fc2)))
        x = _dot(p1ci_r[...], wa_r[...]) + _dot(p1cj_r[...], wb_r[...]) + b_r[...]
        iw = jnp.tanh(x) * basis_t
        i1_1 = _dot(iw, w2a_r[...])
        i1_3 = _dot(iw, w2c_r[...])
        o2_r[...] = _dot(iw, w2b_r[...])
        for c, o3_r in enumerate((o30_r, o31_r, o32_r)):
            v = diffb[:, c:c + 1] * i1_1
            if has_p3:
                v = v + p3rs[c][...] * i1_3
            o3_r[...] = v

    ebs = pl.BlockSpec((be, d), lambda i: (i, 0))
    dbs = pl.BlockSpec((be, 3), lambda i: (i, 0))
    wbs = lambda s: pl.BlockSpec(s, lambda i: (0, 0))
    in_specs = [ebs, ebs]
    args = [p1ci, p1cj]
    if has_p3:
        in_specs += [ebs, ebs, ebs]
        args += list(p3j)
    in_specs += [dbs, wbs((d, d * NB)), wbs((d, d * NB)), wbs((1, d * NB)),
                 wbs((d * NB, d)), wbs((d * NB, d)), wbs((d * NB, d))]
    args += [diff, pi_Wa, pi_Wb, pi_b2, W2a, W2b, W2c]
    out = jax.ShapeDtypeStruct((e, d), jnp.float32)
    return pl.pallas_call(
        body,
        grid=(e // be,),
        in_specs=in_specs,
        out_specs=[ebs, ebs, ebs, ebs],
        out_shape=[out, out, out, out],
    )(*args)


def _node_tc_d0(p1, p1n, p3n0, p3n1, p3n2, out_Wd, out_b2, ow2row, pp_W, pp_b2):
    n, d = p1.shape
    bn = 2000
    assert n % bn == 0

    def body(p1_r, p1n_r, a_r, b_r, c_r, ow_r, ob_r, ow2_r, pw_r, pb_r,
             p1o_r, p30_r, p31_r, p32_r, pc_r, o_r):
        a, b, c = a_r[...], b_r[...], c_r[...]
        p1t1 = a * a + b * b + c * c + p1n_r[...]
        p1new = p1_r[...] + p1t1
        p1o_r[...] = p1new
        p30_r[...] = a * p1t1
        p31_r[...] = b * p1t1
        p32_r[...] = c * p1t1
        o = jnp.tanh(_dot(p1new, ow_r[...]) + ob_r[...])
        o_r[...] = jnp.sum(o * ow2_r[...], axis=1, keepdims=True)
        pc_r[...] = jnp.tanh(_dot(p1new, pw_r[...]) + pb_r[...])

    nbs = pl.BlockSpec((bn, d), lambda i: (i, 0))
    obs = pl.BlockSpec((bn, 1), lambda i: (i, 0))
    wbs = lambda s: pl.BlockSpec(s, lambda i: (0, 0))
    nds = jax.ShapeDtypeStruct((n, d), jnp.float32)
    return pl.pallas_call(
        body,
        grid=(n // bn,),
        in_specs=[nbs, nbs, nbs, nbs, nbs,
                  wbs((d, d)), wbs((1, d)), wbs((1, d)), wbs((d, d)), wbs((1, d))],
        out_specs=[nbs, nbs, nbs, nbs, nbs, obs],
        out_shape=[nds, nds, nds, nds, nds,
                   jax.ShapeDtypeStruct((n, 1), jnp.float32)],
    )(p1, p1n, p3n0, p3n1, p3n2, out_Wd, out_b2, ow2row, pp_W, pp_b2)


def _node_tc_d1(p1, p1n, p3n0, p3n1, p3n2, o_prev, out_Wd, out_b2, ow2row):
    n, d = p1.shape
    bn = 2000
    assert n % bn == 0

    def body(p1_r, p1n_r, a_r, b_r, c_r, op_r, ow_r, ob_r, ow2_r, o_r):
        a, b, c = a_r[...], b_r[...], c_r[...]
        p1t1 = a * a + b * b + c * c + p1n_r[...]
        p1new = p1_r[...] + p1t1
        o = jnp.tanh(_dot(p1new, ow_r[...]) + ob_r[...])
        o_r[...] = op_r[...] + jnp.sum(o * ow2_r[...], axis=1, keepdims=True)

    nbs = pl.BlockSpec((bn, d), lambda i: (i, 0))
    obs = pl.BlockSpec((bn, 1), lambda i: (i, 0))
    wbs = lambda s: pl.BlockSpec(s, lambda i: (0, 0))
    return pl.pallas_call(
        body,
        grid=(n // bn,),
        in_specs=[nbs, nbs, nbs, nbs, nbs, obs,
                  wbs((d, d)), wbs((1, d)), wbs((1, d))],
        out_specs=obs,
        out_shape=jax.ShapeDtypeStruct((n, 1), jnp.float32),
    )(p1, p1n, p3n0, p3n1, p3n2, o_prev, out_Wd, out_b2, ow2row)


def _sc_gather(p1c, p3, ind_i, ind_j):
    n, d = p1c.shape
    e = ind_i.shape[0]
    assert e % WIN == 0
    nwin = e // WIN
    niter = (nwin + NWK - 1) // NWK
    has_p3 = p3 is not None
    mesh = plsc.VectorSubcoreMesh(core_axis_name="c", subcore_axis_name="s")

    eds = jax.ShapeDtypeStruct((e, d), jnp.float32)
    out_type = [eds, eds] + ([eds, eds, eds] if has_p3 else [])
    scratch = [pltpu.VMEM((WIN,), jnp.int32), pltpu.VMEM((WIN,), jnp.int32),
               pltpu.VMEM((WIN, d), jnp.float32), pltpu.VMEM((WIN, d), jnp.float32)]
    if has_p3:
        scratch += [pltpu.VMEM((WIN, d), jnp.float32)] * 3
    scratch += [pltpu.SemaphoreType.DMA]

    def body(*refs):
        if has_p3:
            (p1c_h, p30_h, p31_h, p32_h, ii_h, ij_h,
             oi_h, oj_h, o30_h, o31_h, o32_h,
             ivi, ivj, bi, bj, b0, b1, b2, sem) = refs
        else:
            (p1c_h, ii_h, ij_h, oi_h, oj_h, ivi, ivj, bi, bj, sem) = refs
        wid = lax.axis_index("s") * NC + lax.axis_index("c")

        @pl.loop(0, niter)
        def _(t):
            w = t * NWK + wid

            @pl.when(w < nwin)
            def _():
                base = w * WIN
                pltpu.sync_copy(ii_h.at[pl.ds(base, WIN)], ivi)
                pltpu.sync_copy(ij_h.at[pl.ds(base, WIN)], ivj)
                cps = [pltpu.async_copy(p1c_h.at[ivi], bi, sem),
                       pltpu.async_copy(p1c_h.at[ivj], bj, sem)]
                if has_p3:
                    cps += [pltpu.async_copy(p30_h.at[ivj], b0, sem),
                            pltpu.async_copy(p31_h.at[ivj], b1, sem),
                            pltpu.async_copy(p32_h.at[ivj], b2, sem)]
                for cp in cps:
                    cp.wait()
                pltpu.sync_copy(bi, oi_h.at[pl.ds(base, WIN)])
                pltpu.sync_copy(bj, oj_h.at[pl.ds(base, WIN)])
                if has_p3:
                    pltpu.sync_copy(b0, o30_h.at[pl.ds(base, WIN)])
                    pltpu.sync_copy(b1, o31_h.at[pl.ds(base, WIN)])
                    pltpu.sync_copy(b2, o32_h.at[pl.ds(base, WIN)])

    args = [p1c] + (list(p3) if has_p3 else []) + [ind_i, ind_j]
    return pl.kernel(body, out_type=out_type, mesh=mesh,
                     scratch_types=scratch)(*args)


def _sc_scatter(ind_i, v1, i30, i31, i32, n):
    e, d = v1.shape
    assert e % WIN == 0
    nwin = e // WIN
    niter = (nwin + NS - 1) // NS
    rows = n // NS
    zrows = 625
    assert n % NS == 0 and rows % zrows == 0
    mesh = plsc.VectorSubcoreMesh(core_axis_name="c", subcore_axis_name="s")
    nds = jax.ShapeDtypeStruct((n, d), jnp.float32)

    def body(ii_h, v1_h, i30_h, i31_h, i32_h,
             p1n_h, o30_h, o31_h, o32_h,
             accA, accB, zb, iv, va, vb):
        cid = lax.axis_index("c")
        sid = lax.axis_index("s")

        @pl.loop(0, zrows)
        def _(r):
            zb[r, :] = jnp.zeros((d,), jnp.float32)

        @pl.loop(0, rows // zrows)
        def _(k):
            off = sid * rows + k * zrows
            pltpu.sync_copy(zb, accA.at[pl.ds(off, zrows)])
            pltpu.sync_copy(zb, accB.at[pl.ds(off, zrows)])

        plsc.subcore_barrier()

        @pl.loop(0, niter)
        def _(t):
            w = t * NS + sid

            @pl.when(w < nwin)
            def _():
                base = w * WIN
                pltpu.sync_copy(ii_h.at[pl.ds(base, WIN)], iv)

                @pl.when(cid == 0)
                def _():
                    pltpu.sync_copy(v1_h.at[pl.ds(base, WIN)], va)
                    pltpu.sync_copy(i30_h.at[pl.ds(base, WIN)], vb)

                @pl.when(cid == 1)
                def _():
                    pltpu.sync_copy(i31_h.at[pl.ds(base, WIN)], va)
                    pltpu.sync_copy(i32_h.at[pl.ds(base, WIN)], vb)

                pltpu.sync_copy(va, accA.at[iv], add=True)
                pltpu.sync_copy(vb, accB.at[iv], add=True)

        plsc.subcore_barrier()

        off = sid * rows

        @pl.when(cid == 0)
        def _():
            pltpu.sync_copy(accA.at[pl.ds(off, rows)], p1n_h.at[pl.ds(off, rows)])
            pltpu.sync_copy(accB.at[pl.ds(off, rows)], o30_h.at[pl.ds(off, rows)])

        @pl.when(cid == 1)
        def _():
            pltpu.sync_copy(accA.at[pl.ds(off, rows)], o31_h.at[pl.ds(off, rows)])
            pltpu.sync_copy(accB.at[pl.ds(off, rows)], o32_h.at[pl.ds(off, rows)])

    return pl.kernel(
        body,
        out_type=[nds, nds, nds, nds],
        mesh=mesh,
        scratch_types=[
            pltpu.VMEM_SHARED((n, d), jnp.float32),
            pltpu.VMEM_SHARED((n, d), jnp.float32),
            pltpu.VMEM((zrows, d), jnp.float32),
            pltpu.VMEM((WIN,), jnp.int32),
            pltpu.VMEM((WIN, d), jnp.float32),
            pltpu.VMEM((WIN, d), jnp.float32),
        ],
    )(ind_i, v1, i30, i31, i32)


def kernel(prop, diff, ind_2, pp_W, pp_b, pi_W, pi_b, ii_W, out_W, out_b, out_w2):
    n, d = prop.shape
    ind_i = ind_2[:, 0]
    ind_j = ind_2[:, 1]

    p1ci, p1cj = _sc_gather(prop, None, ind_i, ind_j)
    W2 = jnp.repeat(ii_W[0], NB, axis=0)
    i1_2, i30, i31, i32 = _edge_tc(
        p1ci, p1cj, None, diff, pi_W[0][:d], pi_W[0][d:], pi_b[0][None],
        W2[:, :d], W2[:, d:2 * d], W2[:, 2 * d:])
    p1n, p3n0, p3n1, p3n2 = _sc_scatter(ind_i, i1_2, i30, i31, i32, n)
    p1, p30, p31, p32, p1c, o = _node_tc_d0(
        prop, p1n, p3n0, p3n1, p3n2,
        out_W[0], out_b[0][None], out_w2[0].reshape(1, d), pp_W, pp_b[None])

    p1ci, p1cj, p3j0, p3j1, p3j2 = _sc_gather(p1c, (p30, p31, p32), ind_i, ind_j)
    W2 = jnp.repeat(ii_W[1], NB, axis=0)
    i1_2, i30, i31, i32 = _edge_tc(
        p1ci, p1cj, (p3j0, p3j1, p3j2), diff, pi_W[1][:d], pi_W[1][d:],
        pi_b[1][None], W2[:, :d], W2[:, d:2 * d], W2[:, 2 * d:])
    p1n, p3n0, p3n1, p3n2 = _sc_scatter(ind_i, i1_2, i30, i31, i32, n)
    return _node_tc_d1(p1, p1n, p3n0, p3n1, p3n2, o,
                       out_W[1], out_b[1][None], out_w2[1].reshape(1, d))


# speedup vs baseline: 16.2803x; 16.2803x over previous
import jax
import jax.numpy as jnp
from jax import lax
from jax.experimental import pallas as pl
from jax.experimental.pallas import tpu as pltpu
from jax.experimental.pallas import tpu_sc as plsc

RC = 5.0
NB = 4
NC = 2
NS = 16
NWK = NC * NS
WIN = 128

def _dot(a, b):
    return jnp.dot(a, b, preferred_element_type=jnp.float32)


def _edge_tc(p1ci, p1cj, p3j, diff, pi_Wa, pi_Wb, pi_b2, S, W2a, W2b, W2c):
    e, d = p1ci.shape
    be = 2000
    assert e % be == 0
    has_p3 = p3j is not None

    def body(*refs):
        if has_p3:
            (p1ci_r, p1cj_r, p3j0_r, p3j1_r, p3j2_r, diff_r,
             wa_r, wb_r, b_r, s_r, w2a_r, w2b_r, w2c_r,
             o2_r, o30_r, o31_r, o32_r) = refs
            p3rs = (p3j0_r, p3j1_r, p3j2_r)
        else:
            (p1ci_r, p1cj_r, diff_r,
             wa_r, wb_r, b_r, s_r, w2a_r, w2b_r, w2c_r,
             o2_r, o30_r, o31_r, o32_r) = refs
        diffb = diff_r[...]
        dist = jnp.sqrt(jnp.sum(diffb * diffb, axis=1, keepdims=True) + 1e-12)
        fc = jnp.where(dist < RC, 0.5 * (jnp.cos(jnp.pi / RC * dist) + 1.0), 0.0)
        col = lax.broadcasted_iota(jnp.int32, (be, d * NB), 1) % NB
        fcb = jnp.broadcast_to(fc, (be, d * NB))
        fc2 = fcb * fcb
        basis_t = jnp.where(col == 0, fcb,
                  jnp.where(col == 1, fc2,
                  jnp.where(col == 2, fc2 * fcb, fc2 * fc2)))
        x = _dot(p1ci_r[...], wa_r[...]) + _dot(p1cj_r[...], wb_r[...]) + b_r[...]
        iw = jnp.tanh(x) * basis_t
        i1 = jnp.dot(iw, s_r[...], precision=lax.Precision.HIGHEST,
                     preferred_element_type=jnp.float32)
        i1_1 = _dot(i1, w2a_r[...])
        i1_3 = _dot(i1, w2c_r[...])
        o2_r[...] = _dot(i1, w2b_r[...])
        for c, o3_r in enumerate((o30_r, o31_r, o32_r)):
            v = diffb[:, c:c + 1] * i1_1
            if has_p3:
                v = v + p3rs[c][...] * i1_3
            o3_r[...] = v

    ebs = pl.BlockSpec((be, d), lambda i: (i, 0))
    dbs = pl.BlockSpec((be, 3), lambda i: (i, 0))
    wbs = lambda s: pl.BlockSpec(s, lambda i: (0, 0))
    in_specs = [ebs, ebs]
    args = [p1ci, p1cj]
    if has_p3:
        in_specs += [ebs, ebs, ebs]
        args += list(p3j)
    in_specs += [dbs, wbs((d, d * NB)), wbs((d, d * NB)), wbs((1, d * NB)),
                 wbs((d * NB, d)), wbs((d, d)), wbs((d, d)), wbs((d, d))]
    args += [diff, pi_Wa, pi_Wb, pi_b2, S, W2a, W2b, W2c]
    out = jax.ShapeDtypeStruct((e, d), jnp.float32)
    return pl.pallas_call(
        body,
        grid=(e // be,),
        in_specs=in_specs,
        out_specs=[ebs, ebs, ebs, ebs],
        out_shape=[out, out, out, out],
    )(*args)


def _node_tc_d0(p1, p1n, p3n0, p3n1, p3n2, out_Wd, out_b2, ow2col, pp_W, pp_b2):
    n, d = p1.shape
    bn = 2000
    assert n % bn == 0

    def body(p1_r, p1n_r, a_r, b_r, c_r, ow_r, ob_r, ow2_r, pw_r, pb_r,
             p1o_r, p30_r, p31_r, p32_r, pc_r, o_r):
        a, b, c = a_r[...], b_r[...], c_r[...]
        p1t1 = a * a + b * b + c * c + p1n_r[...]
        p1new = p1_r[...] + p1t1
        p1o_r[...] = p1new
        p30_r[...] = a * p1t1
        p31_r[...] = b * p1t1
        p32_r[...] = c * p1t1
        o = jnp.tanh(_dot(p1new, ow_r[...]) + ob_r[...])
        o_r[...] = _dot(o, ow2_r[...])
        pc_r[...] = jnp.tanh(_dot(p1new, pw_r[...]) + pb_r[...])

    nbs = pl.BlockSpec((bn, d), lambda i: (i, 0))
    obs = pl.BlockSpec((bn, 1), lambda i: (i, 0))
    wbs = lambda s: pl.BlockSpec(s, lambda i: (0, 0))
    nds = jax.ShapeDtypeStruct((n, d), jnp.float32)
    return pl.pallas_call(
        body,
        grid=(n // bn,),
        in_specs=[nbs, nbs, nbs, nbs, nbs,
                  wbs((d, d)), wbs((1, d)), wbs((d, 1)), wbs((d, d)), wbs((1, d))],
        out_specs=[nbs, nbs, nbs, nbs, nbs, obs],
        out_shape=[nds, nds, nds, nds, nds,
                   jax.ShapeDtypeStruct((n, 1), jnp.float32)],
    )(p1, p1n, p3n0, p3n1, p3n2, out_Wd, out_b2, ow2col, pp_W, pp_b2)


def _node_tc_d1(p1, p1n, p3n0, p3n1, p3n2, o_prev, out_Wd, out_b2, ow2col):
    n, d = p1.shape
    bn = 2000
    assert n % bn == 0

    def body(p1_r, p1n_r, a_r, b_r, c_r, op_r, ow_r, ob_r, ow2_r, o_r):
        a, b, c = a_r[...], b_r[...], c_r[...]
        p1t1 = a * a + b * b + c * c + p1n_r[...]
        p1new = p1_r[...] + p1t1
        o = jnp.tanh(_dot(p1new, ow_r[...]) + ob_r[...])
        o_r[...] = op_r[...] + _dot(o, ow2_r[...])

    nbs = pl.BlockSpec((bn, d), lambda i: (i, 0))
    obs = pl.BlockSpec((bn, 1), lambda i: (i, 0))
    wbs = lambda s: pl.BlockSpec(s, lambda i: (0, 0))
    return pl.pallas_call(
        body,
        grid=(n // bn,),
        in_specs=[nbs, nbs, nbs, nbs, nbs, obs,
                  wbs((d, d)), wbs((1, d)), wbs((d, 1))],
        out_specs=obs,
        out_shape=jax.ShapeDtypeStruct((n, 1), jnp.float32),
    )(p1, p1n, p3n0, p3n1, p3n2, o_prev, out_Wd, out_b2, ow2col)


def _sc_gather(p1c, p3, ind_i, ind_j):
    n, d = p1c.shape
    e = ind_i.shape[0]
    assert e % WIN == 0
    nwin = e // WIN
    niter = (nwin + NWK - 1) // NWK
    has_p3 = p3 is not None
    mesh = plsc.VectorSubcoreMesh(core_axis_name="c", subcore_axis_name="s")

    eds = jax.ShapeDtypeStruct((e, d), jnp.float32)
    out_type = [eds, eds] + ([eds, eds, eds] if has_p3 else [])
    scratch = [pltpu.VMEM((WIN,), jnp.int32), pltpu.VMEM((WIN,), jnp.int32),
               pltpu.VMEM((WIN, d), jnp.float32), pltpu.VMEM((WIN, d), jnp.float32)]
    if has_p3:
        scratch += [pltpu.VMEM((WIN, d), jnp.float32)] * 3
    scratch += [pltpu.SemaphoreType.DMA]

    def body(*refs):
        if has_p3:
            (p1c_h, p30_h, p31_h, p32_h, ii_h, ij_h,
             oi_h, oj_h, o30_h, o31_h, o32_h,
             ivi, ivj, bi, bj, b0, b1, b2, sem) = refs
        else:
            (p1c_h, ii_h, ij_h, oi_h, oj_h, ivi, ivj, bi, bj, sem) = refs
        wid = lax.axis_index("s") * NC + lax.axis_index("c")

        @pl.loop(0, niter)
        def _(t):
            w = t * NWK + wid

            @pl.when(w < nwin)
            def _():
                base = w * WIN
                pltpu.sync_copy(ii_h.at[pl.ds(base, WIN)], ivi)
                pltpu.sync_copy(ij_h.at[pl.ds(base, WIN)], ivj)
                cps = [pltpu.async_copy(p1c_h.at[ivi], bi, sem),
                       pltpu.async_copy(p1c_h.at[ivj], bj, sem)]
                if has_p3:
                    cps += [pltpu.async_copy(p30_h.at[ivj], b0, sem),
                            pltpu.async_copy(p31_h.at[ivj], b1, sem),
                            pltpu.async_copy(p32_h.at[ivj], b2, sem)]
                for cp in cps:
                    cp.wait()
                pltpu.sync_copy(bi, oi_h.at[pl.ds(base, WIN)])
                pltpu.sync_copy(bj, oj_h.at[pl.ds(base, WIN)])
                if has_p3:
                    pltpu.sync_copy(b0, o30_h.at[pl.ds(base, WIN)])
                    pltpu.sync_copy(b1, o31_h.at[pl.ds(base, WIN)])
                    pltpu.sync_copy(b2, o32_h.at[pl.ds(base, WIN)])

    args = [p1c] + (list(p3) if has_p3 else []) + [ind_i, ind_j]
    return pl.kernel(body, out_type=out_type, mesh=mesh,
                     scratch_types=scratch,
                     compiler_params=pltpu.CompilerParams(
                         use_tc_tiling_on_sc=False))(*args)


def _sc_scatter(ind_i, v1, i30, i31, i32, n):
    e, d = v1.shape
    assert e % WIN == 0
    nwin = e // WIN
    niter = (nwin + NS - 1) // NS
    rows = n // NS
    zrows = 625
    assert n % NS == 0 and rows % zrows == 0
    mesh = plsc.VectorSubcoreMesh(core_axis_name="c", subcore_axis_name="s")
    nds = jax.ShapeDtypeStruct((n, d), jnp.float32)

    def body(ii_h, v1_h, i30_h, i31_h, i32_h,
             p1n_h, o30_h, o31_h, o32_h,
             accA, accB, zb, iv, va, vb):
        cid = lax.axis_index("c")
        sid = lax.axis_index("s")

        @pl.loop(0, zrows)
        def _(r):
            zb[r, :] = jnp.zeros((d,), jnp.float32)

        @pl.loop(0, rows // zrows)
        def _(k):
            off = sid * rows + k * zrows
            pltpu.sync_copy(zb, accA.at[pl.ds(off, zrows)])
            pltpu.sync_copy(zb, accB.at[pl.ds(off, zrows)])

        plsc.subcore_barrier()

        @pl.loop(0, niter)
        def _(t):
            w = t * NS + sid

            @pl.when(w < nwin)
            def _():
                base = w * WIN
                pltpu.sync_copy(ii_h.at[pl.ds(base, WIN)], iv)

                @pl.when(cid == 0)
                def _():
                    pltpu.sync_copy(v1_h.at[pl.ds(base, WIN)], va)
                    pltpu.sync_copy(i30_h.at[pl.ds(base, WIN)], vb)

                @pl.when(cid == 1)
                def _():
                    pltpu.sync_copy(i31_h.at[pl.ds(base, WIN)], va)
                    pltpu.sync_copy(i32_h.at[pl.ds(base, WIN)], vb)

                pltpu.sync_copy(va, accA.at[iv], add=True)
                pltpu.sync_copy(vb, accB.at[iv], add=True)

        plsc.subcore_barrier()

        off = sid * rows

        @pl.when(cid == 0)
        def _():
            pltpu.sync_copy(accA.at[pl.ds(off, rows)], p1n_h.at[pl.ds(off, rows)])
            pltpu.sync_copy(accB.at[pl.ds(off, rows)], o30_h.at[pl.ds(off, rows)])

        @pl.when(cid == 1)
        def _():
            pltpu.sync_copy(accA.at[pl.ds(off, rows)], o31_h.at[pl.ds(off, rows)])
            pltpu.sync_copy(accB.at[pl.ds(off, rows)], o32_h.at[pl.ds(off, rows)])

    return pl.kernel(
        body,
        out_type=[nds, nds, nds, nds],
        mesh=mesh,
        scratch_types=[
            pltpu.VMEM_SHARED((n, d), jnp.float32),
            pltpu.VMEM_SHARED((n, d), jnp.float32),
            pltpu.VMEM((zrows, d), jnp.float32),
            pltpu.VMEM((WIN,), jnp.int32),
            pltpu.VMEM((WIN, d), jnp.float32),
            pltpu.VMEM((WIN, d), jnp.float32),
        ],
        compiler_params=pltpu.CompilerParams(use_tc_tiling_on_sc=False),
    )(ind_i, v1, i30, i31, i32)


def kernel(prop, diff, ind_2, pp_W, pp_b, pi_W, pi_b, ii_W, out_W, out_b, out_w2):
    n, d = prop.shape
    ind_i = ind_2[:, 0]
    ind_j = ind_2[:, 1]

    S = jnp.repeat(jnp.eye(d, dtype=jnp.float32), NB, axis=0)

    p1ci, p1cj = _sc_gather(prop, None, ind_i, ind_j)
    i1_2, i30, i31, i32 = _edge_tc(
        p1ci, p1cj, None, diff, pi_W[0][:d], pi_W[0][d:], pi_b[0][None],
        S, ii_W[0][:, :d], ii_W[0][:, d:2 * d], ii_W[0][:, 2 * d:])
    p1n, p3n0, p3n1, p3n2 = _sc_scatter(ind_i, i1_2, i30, i31, i32, n)
    p1, p30, p31, p32, p1c, o = _node_tc_d0(
        prop, p1n, p3n0, p3n1, p3n2,
        out_W[0], out_b[0][None], out_w2[0], pp_W, pp_b[None])

    p1ci, p1cj, p3j0, p3j1, p3j2 = _sc_gather(p1c, (p30, p31, p32), ind_i, ind_j)
    i1_2, i30, i31, i32 = _edge_tc(
        p1ci, p1cj, (p3j0, p3j1, p3j2), diff, pi_W[1][:d], pi_W[1][d:],
        pi_b[1][None], S, ii_W[1][:, :d], ii_W[1][:, d:2 * d], ii_W[1][:, 2 * d:])
    p1n, p3n0, p3n1, p3n2 = _sc_scatter(ind_i, i1_2, i30, i31, i32, n)
    return _node_tc_d1(p1, p1n, p3n0, p3n1, p3n2, o,
                       out_W[1], out_b[1][None], out_w2[1])


# speedup vs baseline: 20.5448x; 1.2619x over previous
import jax
import jax.numpy as jnp
from jax import lax
from jax.experimental import pallas as pl
from jax.experimental.pallas import tpu as pltpu
from jax.experimental.pallas import tpu_sc as plsc

RC = 5.0
NB = 4
NC = 2
NS = 16
NWK = NC * NS
WIN = 128

def _dot(a, b):
    return jnp.dot(a, b, preferred_element_type=jnp.float32)


def _cutoff_tc(d0, d1_, d2):
    rows = d0.shape[0]

    def body(x_r, y_r, z_r, o_r):
        x, y, z = x_r[...], y_r[...], z_r[...]
        dist = jnp.sqrt(x * x + y * y + z * z + 1e-12)
        o_r[...] = 0.5 * jnp.cos(jnp.pi / RC * jnp.minimum(dist, RC)) + 0.5

    bs = pl.BlockSpec((rows, 128), lambda: (0, 0))
    return pl.pallas_call(
        body, in_specs=[bs, bs, bs], out_specs=bs,
        out_shape=jax.ShapeDtypeStruct((rows, 128), jnp.float32),
    )(d0, d1_, d2)


def _edge_tc(p1ci, p1cj, p3j, diff, fc, pi_Wa, pi_Wb, pi_b2, W2a, W2b, W2c):
    e, d = p1ci.shape
    be = 2000
    assert e % be == 0
    has_p3 = p3j is not None

    def body(*refs):
        if has_p3:
            (p1ci_r, p1cj_r, p3j0_r, p3j1_r, p3j2_r, diff_r, fc_r,
             wa_r, wb_r, b_r, w2a_r, w2b_r, w2c_r,
             o2_r, o30_r, o31_r, o32_r) = refs
            p3rs = (p3j0_r, p3j1_r, p3j2_r)
        else:
            (p1ci_r, p1cj_r, diff_r, fc_r,
             wa_r, wb_r, b_r, w2a_r, w2b_r, w2c_r,
             o2_r, o30_r, o31_r, o32_r) = refs
        diffb = diff_r[...]
        dxyz = (diffb[:, 0:1], diffb[:, 1:2], diffb[:, 2:3])
        fcb = jnp.broadcast_to(fc_r[...], (be, d))
        fc2b = fcb * fcb
        x = _dot(p1ci_r[...], wa_r[...]) + _dot(p1cj_r[...], wb_r[...]) + b_r[...]
        inter = jnp.tanh(x)
        i1 = (inter[:, 0:d] * fcb + inter[:, d:2 * d] * fc2b
              + inter[:, 2 * d:3 * d] * (fc2b * fcb)
              + inter[:, 3 * d:4 * d] * (fc2b * fc2b))
        i1_1 = _dot(i1, w2a_r[...])
        i1_3 = _dot(i1, w2c_r[...])
        o2_r[...] = _dot(i1, w2b_r[...])
        for c, o3_r in enumerate((o30_r, o31_r, o32_r)):
            v = dxyz[c] * i1_1
            if has_p3:
                v = v + p3rs[c][...] * i1_3
            o3_r[...] = v

    ebs = pl.BlockSpec((be, d), lambda i: (i, 0))
    dbs = pl.BlockSpec((be, 3), lambda i: (i, 0))
    wbs = lambda s: pl.BlockSpec(s, lambda i: (0, 0))
    in_specs = [ebs, ebs]
    args = [p1ci, p1cj]
    if has_p3:
        in_specs += [ebs, ebs, ebs]
        args += list(p3j)
    in_specs += [dbs, pl.BlockSpec((be, 1), lambda i: (i, 0)),
                 wbs((d, d * NB)), wbs((d, d * NB)), wbs((1, d * NB)),
                 wbs((d, d)), wbs((d, d)), wbs((d, d))]
    args += [diff, fc, pi_Wa, pi_Wb, pi_b2, W2a, W2b, W2c]
    out = jax.ShapeDtypeStruct((e, d), jnp.float32)
    return pl.pallas_call(
        body,
        grid=(e // be,),
        in_specs=in_specs,
        out_specs=[ebs, ebs, ebs, ebs],
        out_shape=[out, out, out, out],
    )(*args)


def _node_tc_d0(p1, p1n, p3n0, p3n1, p3n2, out_Wd, out_b2, ow2col, pp_W, pp_b2):
    n, d = p1.shape
    bn = 2000
    assert n % bn == 0

    def body(p1_r, p1n_r, a_r, b_r, c_r, ow_r, ob_r, ow2_r, pw_r, pb_r,
             p1o_r, p30_r, p31_r, p32_r, pc_r, o_r):
        a, b, c = a_r[...], b_r[...], c_r[...]
        p1t1 = a * a + b * b + c * c + p1n_r[...]
        p1new = p1_r[...] + p1t1
        p1o_r[...] = p1new
        p30_r[...] = a * p1t1
        p31_r[...] = b * p1t1
        p32_r[...] = c * p1t1
        o = jnp.tanh(_dot(p1new, ow_r[...]) + ob_r[...])
        o_r[...] = _dot(o, ow2_r[...])
        pc_r[...] = jnp.tanh(_dot(p1new, pw_r[...]) + pb_r[...])

    nbs = pl.BlockSpec((bn, d), lambda i: (i, 0))
    obs = pl.BlockSpec((bn, 1), lambda i: (i, 0))
    wbs = lambda s: pl.BlockSpec(s, lambda i: (0, 0))
    nds = jax.ShapeDtypeStruct((n, d), jnp.float32)
    return pl.pallas_call(
        body,
        grid=(n // bn,),
        in_specs=[nbs, nbs, nbs, nbs, nbs,
                  wbs((d, d)), wbs((1, d)), wbs((d, 1)), wbs((d, d)), wbs((1, d))],
        out_specs=[nbs, nbs, nbs, nbs, nbs, obs],
        out_shape=[nds, nds, nds, nds, nds,
                   jax.ShapeDtypeStruct((n, 1), jnp.float32)],
    )(p1, p1n, p3n0, p3n1, p3n2, out_Wd, out_b2, ow2col, pp_W, pp_b2)


def _node_tc_d1(p1, p1n, p3n0, p3n1, p3n2, o_prev, out_Wd, out_b2, ow2col):
    n, d = p1.shape
    bn = 2000
    assert n % bn == 0

    def body(p1_r, p1n_r, a_r, b_r, c_r, op_r, ow_r, ob_r, ow2_r, o_r):
        a, b, c = a_r[...], b_r[...], c_r[...]
        p1t1 = a * a + b * b + c * c + p1n_r[...]
        p1new = p1_r[...] + p1t1
        o = jnp.tanh(_dot(p1new, ow_r[...]) + ob_r[...])
        o_r[...] = op_r[...] + _dot(o, ow2_r[...])

    nbs = pl.BlockSpec((bn, d), lambda i: (i, 0))
    obs = pl.BlockSpec((bn, 1), lambda i: (i, 0))
    wbs = lambda s: pl.BlockSpec(s, lambda i: (0, 0))
    return pl.pallas_call(
        body,
        grid=(n // bn,),
        in_specs=[nbs, nbs, nbs, nbs, nbs, obs,
                  wbs((d, d)), wbs((1, d)), wbs((d, 1))],
        out_specs=obs,
        out_shape=jax.ShapeDtypeStruct((n, 1), jnp.float32),
    )(p1, p1n, p3n0, p3n1, p3n2, o_prev, out_Wd, out_b2, ow2col)


def _sc_gather(p1c, p3, ind_i, ind_j):
    n, d = p1c.shape
    e = ind_i.shape[0]
    assert e % WIN == 0
    nwin = e // WIN
    niter = (nwin + NWK - 1) // NWK
    has_p3 = p3 is not None
    mesh = plsc.VectorSubcoreMesh(core_axis_name="c", subcore_axis_name="s")

    eds = jax.ShapeDtypeStruct((e, d), jnp.float32)
    out_type = [eds, eds] + ([eds, eds, eds] if has_p3 else [])
    scratch = [pltpu.VMEM((WIN,), jnp.int32), pltpu.VMEM((WIN,), jnp.int32),
               pltpu.VMEM((WIN, d), jnp.float32), pltpu.VMEM((WIN, d), jnp.float32)]
    if has_p3:
        scratch += [pltpu.VMEM((WIN, d), jnp.float32)] * 3
    scratch += [pltpu.SemaphoreType.DMA]

    def body(*refs):
        if has_p3:
            (p1c_h, p30_h, p31_h, p32_h, ii_h, ij_h,
             oi_h, oj_h, o30_h, o31_h, o32_h,
             ivi, ivj, bi, bj, b0, b1, b2, sem) = refs
        else:
            (p1c_h, ii_h, ij_h, oi_h, oj_h, ivi, ivj, bi, bj, sem) = refs
        wid = lax.axis_index("s") * NC + lax.axis_index("c")

        @pl.loop(0, niter)
        def _(t):
            w = t * NWK + wid

            @pl.when(w < nwin)
            def _():
                base = w * WIN
                pltpu.sync_copy(ii_h.at[pl.ds(base, WIN)], ivi)
                pltpu.sync_copy(ij_h.at[pl.ds(base, WIN)], ivj)
                cps = [pltpu.async_copy(p1c_h.at[ivi], bi, sem),
                       pltpu.async_copy(p1c_h.at[ivj], bj, sem)]
                if has_p3:
                    cps += [pltpu.async_copy(p30_h.at[ivj], b0, sem),
                            pltpu.async_copy(p31_h.at[ivj], b1, sem),
                            pltpu.async_copy(p32_h.at[ivj], b2, sem)]
                for cp in cps:
                    cp.wait()
                pltpu.sync_copy(bi, oi_h.at[pl.ds(base, WIN)])
                pltpu.sync_copy(bj, oj_h.at[pl.ds(base, WIN)])
                if has_p3:
                    pltpu.sync_copy(b0, o30_h.at[pl.ds(base, WIN)])
                    pltpu.sync_copy(b1, o31_h.at[pl.ds(base, WIN)])
                    pltpu.sync_copy(b2, o32_h.at[pl.ds(base, WIN)])

    args = [p1c] + (list(p3) if has_p3 else []) + [ind_i, ind_j]
    return pl.kernel(body, out_type=out_type, mesh=mesh,
                     scratch_types=scratch,
                     compiler_params=pltpu.CompilerParams(
                         use_tc_tiling_on_sc=False))(*args)


def _sc_scatter(ind_i, v1, i30, i31, i32, n):
    e, d = v1.shape
    assert e % WIN == 0
    nwin = e // WIN
    niter = (nwin + NS - 1) // NS
    rows = n // NS
    zrows = 625
    assert n % NS == 0 and rows % zrows == 0
    mesh = plsc.VectorSubcoreMesh(core_axis_name="c", subcore_axis_name="s")
    nds = jax.ShapeDtypeStruct((n, d), jnp.float32)

    def body(ii_h, v1_h, i30_h, i31_h, i32_h,
             p1n_h, o30_h, o31_h, o32_h,
             accA, accB, zb, iv, va, vb):
        cid = lax.axis_index("c")
        sid = lax.axis_index("s")

        @pl.loop(0, zrows)
        def _(r):
            zb[r, :] = jnp.zeros((d,), jnp.float32)

        @pl.loop(0, rows // zrows)
        def _(k):
            off = sid * rows + k * zrows
            pltpu.sync_copy(zb, accA.at[pl.ds(off, zrows)])
            pltpu.sync_copy(zb, accB.at[pl.ds(off, zrows)])

        plsc.subcore_barrier()

        @pl.loop(0, niter)
        def _(t):
            w = t * NS + sid

            @pl.when(w < nwin)
            def _():
                base = w * WIN
                pltpu.sync_copy(ii_h.at[pl.ds(base, WIN)], iv)

                @pl.when(cid == 0)
                def _():
                    pltpu.sync_copy(v1_h.at[pl.ds(base, WIN)], va)
                    pltpu.sync_copy(i30_h.at[pl.ds(base, WIN)], vb)

                @pl.when(cid == 1)
                def _():
                    pltpu.sync_copy(i31_h.at[pl.ds(base, WIN)], va)
                    pltpu.sync_copy(i32_h.at[pl.ds(base, WIN)], vb)

                pltpu.sync_copy(va, accA.at[iv], add=True)
                pltpu.sync_copy(vb, accB.at[iv], add=True)

        plsc.subcore_barrier()

        off = sid * rows

        @pl.when(cid == 0)
        def _():
            pltpu.sync_copy(accA.at[pl.ds(off, rows)], p1n_h.at[pl.ds(off, rows)])
            pltpu.sync_copy(accB.at[pl.ds(off, rows)], o30_h.at[pl.ds(off, rows)])

        @pl.when(cid == 1)
        def _():
            pltpu.sync_copy(accA.at[pl.ds(off, rows)], o31_h.at[pl.ds(off, rows)])
            pltpu.sync_copy(accB.at[pl.ds(off, rows)], o32_h.at[pl.ds(off, rows)])

    return pl.kernel(
        body,
        out_type=[nds, nds, nds, nds],
        mesh=mesh,
        scratch_types=[
            pltpu.VMEM_SHARED((n, d), jnp.float32),
            pltpu.VMEM_SHARED((n, d), jnp.float32),
            pltpu.VMEM((zrows, d), jnp.float32),
            pltpu.VMEM((WIN,), jnp.int32),
            pltpu.VMEM((WIN, d), jnp.float32),
            pltpu.VMEM((WIN, d), jnp.float32),
        ],
        compiler_params=pltpu.CompilerParams(use_tc_tiling_on_sc=False),
    )(ind_i, v1, i30, i31, i32)


def kernel(prop, diff, ind_2, pp_W, pp_b, pi_W, pi_b, ii_W, out_W, out_b, out_w2):
    n, d = prop.shape
    ind_i = ind_2[:, 0]
    ind_j = ind_2[:, 1]

    perm = jnp.arange(d * NB).reshape(d, NB).T.reshape(-1)
    pi_Wp = pi_W[:, :, perm]
    pi_bp = pi_b[:, perm]

    e = diff.shape[0]
    fc = _cutoff_tc(diff[:, 0].reshape(e // 128, 128),
                    diff[:, 1].reshape(e // 128, 128),
                    diff[:, 2].reshape(e // 128, 128)).reshape(e, 1)

    p1ci, p1cj = _sc_gather(prop, None, ind_i, ind_j)
    i1_2, i30, i31, i32 = _edge_tc(
        p1ci, p1cj, None, diff, fc, pi_Wp[0][:d], pi_Wp[0][d:], pi_bp[0][None],
        ii_W[0][:, :d], ii_W[0][:, d:2 * d], ii_W[0][:, 2 * d:])
    p1n, p3n0, p3n1, p3n2 = _sc_scatter(ind_i, i1_2, i30, i31, i32, n)
    p1, p30, p31, p32, p1c, o = _node_tc_d0(
        prop, p1n, p3n0, p3n1, p3n2,
        out_W[0], out_b[0][None], out_w2[0], pp_W, pp_b[None])

    p1ci, p1cj, p3j0, p3j1, p3j2 = _sc_gather(p1c, (p30, p31, p32), ind_i, ind_j)
    i1_2, i30, i31, i32 = _edge_tc(
        p1ci, p1cj, (p3j0, p3j1, p3j2), diff, fc, pi_Wp[1][:d], pi_Wp[1][d:],
        pi_bp[1][None], ii_W[1][:, :d], ii_W[1][:, d:2 * d], ii_W[1][:, 2 * d:])
    p1n, p3n0, p3n1, p3n2 = _sc_scatter(ind_i, i1_2, i30, i31, i32, n)
    return _node_tc_d1(p1, p1n, p3n0, p3n1, p3n2, o,
                       out_W[1], out_b[1][None], out_w2[1])


# speedup vs baseline: 23.5928x; 1.1484x over previous
import jax
import jax.numpy as jnp
from jax import lax
from jax.experimental import pallas as pl
from jax.experimental.pallas import tpu as pltpu
from jax.experimental.pallas import tpu_sc as plsc

RC = 5.0
NB = 4
NC = 2
NS = 16
NWK = NC * NS
WIN = 400

def _dot(a, b):
    return jnp.dot(a, b, preferred_element_type=jnp.float32)


def _cutoff_tc(d0, d1_, d2):
    rows = d0.shape[0]

    def body(x_r, y_r, z_r, o_r):
        x, y, z = x_r[...], y_r[...], z_r[...]
        dist = jnp.sqrt(x * x + y * y + z * z + 1e-12)
        o_r[...] = 0.5 * jnp.cos(jnp.pi / RC * jnp.minimum(dist, RC)) + 0.5

    bs = pl.BlockSpec((rows, 128), lambda: (0, 0))
    return pl.pallas_call(
        body, in_specs=[bs, bs, bs], out_specs=bs,
        out_shape=jax.ShapeDtypeStruct((rows, 128), jnp.float32),
    )(d0, d1_, d2)


def _edge_tc(p1ci, p1cj, p3j, diff, fc, pi_Wa, pi_Wb, pi_b2, W2a, W2b, W2c):
    e, d = p1ci.shape
    be = 2000
    assert e % be == 0
    has_p3 = p3j is not None

    def body(*refs):
        if has_p3:
            (p1ci_r, p1cj_r, p3j0_r, p3j1_r, p3j2_r, diff_r, fc_r,
             wa_r, wb_r, b_r, w2a_r, w2b_r, w2c_r,
             o2_r, o30_r, o31_r, o32_r) = refs
            p3rs = (p3j0_r, p3j1_r, p3j2_r)
        else:
            (p1ci_r, p1cj_r, diff_r, fc_r,
             wa_r, wb_r, b_r, w2a_r, w2b_r, w2c_r,
             o2_r, o30_r, o31_r, o32_r) = refs
        diffb = diff_r[...]
        dxyz = (diffb[:, 0:1], diffb[:, 1:2], diffb[:, 2:3])
        fcb = jnp.broadcast_to(fc_r[...], (be, d))
        fc2b = fcb * fcb
        x = _dot(p1ci_r[...], wa_r[...]) + _dot(p1cj_r[...], wb_r[...]) + b_r[...]
        inter = jnp.tanh(x)
        i1 = (inter[:, 0:d] * fcb + inter[:, d:2 * d] * fc2b
              + inter[:, 2 * d:3 * d] * (fc2b * fcb)
              + inter[:, 3 * d:4 * d] * (fc2b * fc2b))
        i1_1 = _dot(i1, w2a_r[...])
        i1_3 = _dot(i1, w2c_r[...])
        o2_r[...] = _dot(i1, w2b_r[...])
        for c, o3_r in enumerate((o30_r, o31_r, o32_r)):
            v = dxyz[c] * i1_1
            if has_p3:
                v = v + p3rs[c][...] * i1_3
            o3_r[...] = v

    ebs = pl.BlockSpec((be, d), lambda i: (i, 0))
    dbs = pl.BlockSpec((be, 3), lambda i: (i, 0))
    wbs = lambda s: pl.BlockSpec(s, lambda i: (0, 0))
    in_specs = [ebs, ebs]
    args = [p1ci, p1cj]
    if has_p3:
        in_specs += [ebs, ebs, ebs]
        args += list(p3j)
    in_specs += [dbs, pl.BlockSpec((be, 1), lambda i: (i, 0)),
                 wbs((d, d * NB)), wbs((d, d * NB)), wbs((1, d * NB)),
                 wbs((d, d)), wbs((d, d)), wbs((d, d))]
    args += [diff, fc, pi_Wa, pi_Wb, pi_b2, W2a, W2b, W2c]
    out = jax.ShapeDtypeStruct((e, d), jnp.float32)
    return pl.pallas_call(
        body,
        grid=(e // be,),
        in_specs=in_specs,
        out_specs=[ebs, ebs, ebs, ebs],
        out_shape=[out, out, out, out],
    )(*args)


def _node_tc_d0(p1, p1n, p3n0, p3n1, p3n2, out_Wd, out_b2, ow2col, pp_W, pp_b2):
    n, d = p1.shape
    bn = 2000
    assert n % bn == 0

    def body(p1_r, p1n_r, a_r, b_r, c_r, ow_r, ob_r, ow2_r, pw_r, pb_r,
             p1o_r, p30_r, p31_r, p32_r, pc_r, o_r):
        a, b, c = a_r[...], b_r[...], c_r[...]
        p1t1 = a * a + b * b + c * c + p1n_r[...]
        p1new = p1_r[...] + p1t1
        p1o_r[...] = p1new
        p30_r[...] = a * p1t1
        p31_r[...] = b * p1t1
        p32_r[...] = c * p1t1
        o = jnp.tanh(_dot(p1new, ow_r[...]) + ob_r[...])
        o_r[...] = _dot(o, ow2_r[...])
        pc_r[...] = jnp.tanh(_dot(p1new, pw_r[...]) + pb_r[...])

    nbs = pl.BlockSpec((bn, d), lambda i: (i, 0))
    obs = pl.BlockSpec((bn, 1), lambda i: (i, 0))
    wbs = lambda s: pl.BlockSpec(s, lambda i: (0, 0))
    nds = jax.ShapeDtypeStruct((n, d), jnp.float32)
    return pl.pallas_call(
        body,
        grid=(n // bn,),
        in_specs=[nbs, nbs, nbs, nbs, nbs,
                  wbs((d, d)), wbs((1, d)), wbs((d, 1)), wbs((d, d)), wbs((1, d))],
        out_specs=[nbs, nbs, nbs, nbs, nbs, obs],
        out_shape=[nds, nds, nds, nds, nds,
                   jax.ShapeDtypeStruct((n, 1), jnp.float32)],
    )(p1, p1n, p3n0, p3n1, p3n2, out_Wd, out_b2, ow2col, pp_W, pp_b2)


def _node_tc_d1(p1, p1n, p3n0, p3n1, p3n2, o_prev, out_Wd, out_b2, ow2col):
    n, d = p1.shape
    bn = 2000
    assert n % bn == 0

    def body(p1_r, p1n_r, a_r, b_r, c_r, op_r, ow_r, ob_r, ow2_r, o_r):
        a, b, c = a_r[...], b_r[...], c_r[...]
        p1t1 = a * a + b * b + c * c + p1n_r[...]
        p1new = p1_r[...] + p1t1
        o = jnp.tanh(_dot(p1new, ow_r[...]) + ob_r[...])
        o_r[...] = op_r[...] + _dot(o, ow2_r[...])

    nbs = pl.BlockSpec((bn, d), lambda i: (i, 0))
    obs = pl.BlockSpec((bn, 1), lambda i: (i, 0))
    wbs = lambda s: pl.BlockSpec(s, lambda i: (0, 0))
    return pl.pallas_call(
        body,
        grid=(n // bn,),
        in_specs=[nbs, nbs, nbs, nbs, nbs, obs,
                  wbs((d, d)), wbs((1, d)), wbs((d, 1))],
        out_specs=obs,
        out_shape=jax.ShapeDtypeStruct((n, 1), jnp.float32),
    )(p1, p1n, p3n0, p3n1, p3n2, o_prev, out_Wd, out_b2, ow2col)


def _sc_gather(p1c, p3, ind_i, ind_j):
    n, d = p1c.shape
    e = ind_i.shape[0]
    assert e % WIN == 0
    nwin = e // WIN
    niter = (nwin + NWK - 1) // NWK
    has_p3 = p3 is not None
    mesh = plsc.VectorSubcoreMesh(core_axis_name="c", subcore_axis_name="s")

    eds = jax.ShapeDtypeStruct((e, d), jnp.float32)
    out_type = [eds, eds] + ([eds, eds, eds] if has_p3 else [])
    scratch = [pltpu.VMEM((WIN,), jnp.int32), pltpu.VMEM((WIN,), jnp.int32),
               pltpu.VMEM((WIN, d), jnp.float32), pltpu.VMEM((WIN, d), jnp.float32)]
    if has_p3:
        scratch += [pltpu.VMEM((WIN, d), jnp.float32)] * 3
    scratch += [pltpu.SemaphoreType.DMA]

    def body(*refs):
        if has_p3:
            (p1c_h, p30_h, p31_h, p32_h, ii_h, ij_h,
             oi_h, oj_h, o30_h, o31_h, o32_h,
             ivi, ivj, bi, bj, b0, b1, b2, sem) = refs
        else:
            (p1c_h, ii_h, ij_h, oi_h, oj_h, ivi, ivj, bi, bj, sem) = refs
        wid = lax.axis_index("s") * NC + lax.axis_index("c")

        @pl.loop(0, niter)
        def _(t):
            w = t * NWK + wid

            @pl.when(w < nwin)
            def _():
                base = w * WIN
                pltpu.sync_copy(ii_h.at[pl.ds(base, WIN)], ivi)
                pltpu.sync_copy(ij_h.at[pl.ds(base, WIN)], ivj)
                cps = [pltpu.async_copy(p1c_h.at[ivi], bi, sem),
                       pltpu.async_copy(p1c_h.at[ivj], bj, sem)]
                if has_p3:
                    cps += [pltpu.async_copy(p30_h.at[ivj], b0, sem),
                            pltpu.async_copy(p31_h.at[ivj], b1, sem),
                            pltpu.async_copy(p32_h.at[ivj], b2, sem)]
                for cp in cps:
                    cp.wait()
                pltpu.sync_copy(bi, oi_h.at[pl.ds(base, WIN)])
                pltpu.sync_copy(bj, oj_h.at[pl.ds(base, WIN)])
                if has_p3:
                    pltpu.sync_copy(b0, o30_h.at[pl.ds(base, WIN)])
                    pltpu.sync_copy(b1, o31_h.at[pl.ds(base, WIN)])
                    pltpu.sync_copy(b2, o32_h.at[pl.ds(base, WIN)])

    args = [p1c] + (list(p3) if has_p3 else []) + [ind_i, ind_j]
    return pl.kernel(body, out_type=out_type, mesh=mesh,
                     scratch_types=scratch,
                     compiler_params=pltpu.CompilerParams(
                         use_tc_tiling_on_sc=False))(*args)


def _sc_scatter(ind_i, v1, i30, i31, i32, n):
    e, d = v1.shape
    assert e % WIN == 0
    nwin = e // WIN
    niter = (nwin + NS - 1) // NS
    rows = n // NS
    zrows = 625
    assert n % NS == 0 and rows % zrows == 0
    mesh = plsc.VectorSubcoreMesh(core_axis_name="c", subcore_axis_name="s")
    nds = jax.ShapeDtypeStruct((n, d), jnp.float32)

    def body(ii_h, v1_h, i30_h, i31_h, i32_h,
             p1n_h, o30_h, o31_h, o32_h,
             accA, accB, zb, iv, va, vb):
        cid = lax.axis_index("c")
        sid = lax.axis_index("s")

        @pl.loop(0, zrows)
        def _(r):
            zb[r, :] = jnp.zeros((d,), jnp.float32)

        @pl.loop(0, rows // zrows)
        def _(k):
            off = sid * rows + k * zrows
            pltpu.sync_copy(zb, accA.at[pl.ds(off, zrows)])
            pltpu.sync_copy(zb, accB.at[pl.ds(off, zrows)])

        plsc.subcore_barrier()

        @pl.loop(0, niter)
        def _(t):
            w = t * NS + sid

            @pl.when(w < nwin)
            def _():
                base = w * WIN
                pltpu.sync_copy(ii_h.at[pl.ds(base, WIN)], iv)

                @pl.when(cid == 0)
                def _():
                    pltpu.sync_copy(v1_h.at[pl.ds(base, WIN)], va)
                    pltpu.sync_copy(i30_h.at[pl.ds(base, WIN)], vb)

                @pl.when(cid == 1)
                def _():
                    pltpu.sync_copy(i31_h.at[pl.ds(base, WIN)], va)
                    pltpu.sync_copy(i32_h.at[pl.ds(base, WIN)], vb)

                pltpu.sync_copy(va, accA.at[iv], add=True)
                pltpu.sync_copy(vb, accB.at[iv], add=True)

        plsc.subcore_barrier()

        off = sid * rows

        @pl.when(cid == 0)
        def _():
            pltpu.sync_copy(accA.at[pl.ds(off, rows)], p1n_h.at[pl.ds(off, rows)])
            pltpu.sync_copy(accB.at[pl.ds(off, rows)], o30_h.at[pl.ds(off, rows)])

        @pl.when(cid == 1)
        def _():
            pltpu.sync_copy(accA.at[pl.ds(off, rows)], o31_h.at[pl.ds(off, rows)])
            pltpu.sync_copy(accB.at[pl.ds(off, rows)], o32_h.at[pl.ds(off, rows)])

    return pl.kernel(
        body,
        out_type=[nds, nds, nds, nds],
        mesh=mesh,
        scratch_types=[
            pltpu.VMEM_SHARED((n, d), jnp.float32),
            pltpu.VMEM_SHARED((n, d), jnp.float32),
            pltpu.VMEM((zrows, d), jnp.float32),
            pltpu.VMEM((WIN,), jnp.int32),
            pltpu.VMEM((WIN, d), jnp.float32),
            pltpu.VMEM((WIN, d), jnp.float32),
        ],
        compiler_params=pltpu.CompilerParams(use_tc_tiling_on_sc=False),
    )(ind_i, v1, i30, i31, i32)


def kernel(prop, diff, ind_2, pp_W, pp_b, pi_W, pi_b, ii_W, out_W, out_b, out_w2):
    n, d = prop.shape
    ind_i = ind_2[:, 0]
    ind_j = ind_2[:, 1]

    perm = jnp.arange(d * NB).reshape(d, NB).T.reshape(-1)
    pi_Wp = pi_W[:, :, perm]
    pi_bp = pi_b[:, perm]

    e = diff.shape[0]
    fc = _cutoff_tc(diff[:, 0].reshape(e // 128, 128),
                    diff[:, 1].reshape(e // 128, 128),
                    diff[:, 2].reshape(e // 128, 128)).reshape(e, 1)

    p1ci, p1cj = _sc_gather(prop, None, ind_i, ind_j)
    i1_2, i30, i31, i32 = _edge_tc(
        p1ci, p1cj, None, diff, fc, pi_Wp[0][:d], pi_Wp[0][d:], pi_bp[0][None],
        ii_W[0][:, :d], ii_W[0][:, d:2 * d], ii_W[0][:, 2 * d:])
    p1n, p3n0, p3n1, p3n2 = _sc_scatter(ind_i, i1_2, i30, i31, i32, n)
    p1, p30, p31, p32, p1c, o = _node_tc_d0(
        prop, p1n, p3n0, p3n1, p3n2,
        out_W[0], out_b[0][None], out_w2[0], pp_W, pp_b[None])

    p1ci, p1cj, p3j0, p3j1, p3j2 = _sc_gather(p1c, (p30, p31, p32), ind_i, ind_j)
    i1_2, i30, i31, i32 = _edge_tc(
        p1ci, p1cj, (p3j0, p3j1, p3j2), diff, fc, pi_Wp[1][:d], pi_Wp[1][d:],
        pi_bp[1][None], ii_W[1][:, :d], ii_W[1][:, d:2 * d], ii_W[1][:, 2 * d:])
    p1n, p3n0, p3n1, p3n2 = _sc_scatter(ind_i, i1_2, i30, i31, i32, n)
    return _node_tc_d1(p1, p1n, p3n0, p3n1, p3n2, o,
                       out_W[1], out_b[1][None], out_w2[1])
